# baseline (speedup 1.0000x reference)
import jax
import jax.numpy as jnp
from jax import lax
from jax.experimental import pallas as pl
from jax.experimental.pallas import tpu as pltpu
from jax.experimental.pallas import tpu_sc as plsc

_N = 10000
_D = 128
_NPAD = 10240
_E = 320000
_NC = 2
_NS = 16
_NW = _NC * _NS
_CHUNK = 128
_CPT = 80
_CPQ = 16
_QN = 5
_EPAD = _NW * _CPT * _CHUNK
_RPT = _NPAD // _NS

import functools


@functools.cache
def _mesh():
    return plsc.VectorSubcoreMesh(
        core_axis_name="c", subcore_axis_name="s", num_cores=_NC, num_subcores=_NS
    )


def _sc_degree_body(src_hbm, dst_hbm, zero_hbm, ones_hbm, outs_hbm, outd_hbm,
                    srcv, dstv, onesv, accs, accd, ssem0, ssem1):
    cid = lax.axis_index("c")
    sid = lax.axis_index("s")
    wid = sid * _NC + cid
    r0 = sid * _RPT
    pltpu.sync_copy(zero_hbm.at[pl.ds(r0, _RPT)], accs.at[pl.ds(r0, _RPT)])
    pltpu.sync_copy(zero_hbm.at[pl.ds(r0, _RPT)], accd.at[pl.ds(r0, _RPT)])
    pltpu.sync_copy(ones_hbm, onesv)
    pltpu.sync_copy(src_hbm.at[wid], srcv)
    pltpu.sync_copy(dst_hbm.at[wid], dstv)
    plsc.subcore_barrier()

    def body(j, carry):
        pltpu.async_copy(onesv, accs.at[srcv.at[j]], ssem0, add=True)
        pltpu.async_copy(onesv, accd.at[dstv.at[j]], ssem1, add=True)
        return carry

    lax.fori_loop(0, _CPT, body, 0)

    def drain(j, carry):
        pltpu.make_async_copy(onesv, accs.at[srcv.at[j]], ssem0).wait()
        pltpu.make_async_copy(onesv, accd.at[dstv.at[j]], ssem1).wait()
        return carry

    lax.fori_loop(0, _CPT, drain, 0)
    plsc.subcore_barrier()
    pltpu.sync_copy(accs.at[pl.ds(r0, _RPT)], outs_hbm.at[cid, pl.ds(r0, _RPT)])
    pltpu.sync_copy(accd.at[pl.ds(r0, _RPT)], outd_hbm.at[cid, pl.ds(r0, _RPT)])


@functools.cache
def _degree_kernel():
    return pl.kernel(
        _sc_degree_body,
        out_type=(
            jax.ShapeDtypeStruct((_NC, _NPAD, 16), jnp.float32),
            jax.ShapeDtypeStruct((_NC, _NPAD, 16), jnp.float32),
        ),
        mesh=_mesh(),
        scratch_types=[
            pltpu.VMEM((_CPT, _CHUNK), jnp.int32),
            pltpu.VMEM((_CPT, _CHUNK), jnp.int32),
            pltpu.VMEM((_CHUNK, 16), jnp.float32),
            pltpu.VMEM_SHARED((_NPAD, 16), jnp.float32),
            pltpu.VMEM_SHARED((_NPAD, 16), jnp.float32),
            pltpu.SemaphoreType.DMA,
            pltpu.SemaphoreType.DMA,
        ],
        compiler_params=pltpu.CompilerParams(use_tc_tiling_on_sc=False),
    )


def _sc_scatter_body(m_hbm, src_hbm, dst_hbm, zero_hbm, out_hbm,
                     srcv, dstv, rows0, rows1, acc,
                     gsem0, gsem1, ssem0, ssem1, isem0, isem1):
    cid = lax.axis_index("c")
    sid = lax.axis_index("s")
    wid = sid * _NC + cid
    r0 = sid * _RPT
    pltpu.sync_copy(zero_hbm.at[pl.ds(r0, _RPT)], acc.at[pl.ds(r0, _RPT)])
    pltpu.sync_copy(src_hbm.at[wid, pl.ds(0, _CPQ)], srcv.at[0])
    pltpu.sync_copy(dst_hbm.at[wid, pl.ds(0, _CPQ)], dstv.at[0])
    plsc.subcore_barrier()

    for q in range(_QN):
        p = q % 2
        if q > 0:
            pltpu.make_async_copy(
                src_hbm.at[wid, pl.ds(q * _CPQ, _CPQ)], srcv.at[p], isem0).wait()
            pltpu.make_async_copy(
                dst_hbm.at[wid, pl.ds(q * _CPQ, _CPQ)], dstv.at[p], isem1).wait()
        if q + 1 < _QN:
            pltpu.async_copy(
                src_hbm.at[wid, pl.ds((q + 1) * _CPQ, _CPQ)], srcv.at[1 - p], isem0)
            pltpu.async_copy(
                dst_hbm.at[wid, pl.ds((q + 1) * _CPQ, _CPQ)], dstv.at[1 - p], isem1)
        pltpu.async_copy(m_hbm.at[srcv.at[p, 0]], rows0, gsem0)

        def body(k, carry, p=p):
            j0 = 2 * k
            j1 = j0 + 1
            pltpu.make_async_copy(m_hbm.at[srcv.at[p, j0]], rows0, gsem0).wait()

            @pl.when(k > 0)
            def _():
                pltpu.make_async_copy(rows1, acc.at[dstv.at[p, j1 - 2]], ssem1).wait()

            pltpu.async_copy(m_hbm.at[srcv.at[p, j1]], rows1, gsem1)
            pltpu.async_copy(rows0, acc.at[dstv.at[p, j0]], ssem0, add=True)
            pltpu.make_async_copy(m_hbm.at[srcv.at[p, j1]], rows1, gsem1).wait()
            pltpu.make_async_copy(rows0, acc.at[dstv.at[p, j0]], ssem0).wait()

            @pl.when(k < _CPQ // 2 - 1)
            def _():
                pltpu.async_copy(m_hbm.at[srcv.at[p, j0 + 2]], rows0, gsem0)

            pltpu.async_copy(rows1, acc.at[dstv.at[p, j1]], ssem1, add=True)
            return carry

        lax.fori_loop(0, _CPQ // 2, body, 0)
        pltpu.make_async_copy(rows1, acc.at[dstv.at[p, _CPQ - 1]], ssem1).wait()

    plsc.subcore_barrier()
    pltpu.sync_copy(acc.at[pl.ds(r0, _RPT)], out_hbm.at[cid, pl.ds(r0, _RPT)])


@functools.cache
def _scatter_kernel():
    return pl.kernel(
        _sc_scatter_body,
        out_type=jax.ShapeDtypeStruct((_NC, _NPAD, _D), jnp.float32),
        mesh=_mesh(),
        scratch_types=[
            pltpu.VMEM((2, _CPQ, _CHUNK), jnp.int32),
            pltpu.VMEM((2, _CPQ, _CHUNK), jnp.int32),
            pltpu.VMEM((_CHUNK, _D), jnp.float32),
            pltpu.VMEM((_CHUNK, _D), jnp.float32),
            pltpu.VMEM_SHARED((_NPAD, _D), jnp.float32),
            pltpu.SemaphoreType.DMA,
            pltpu.SemaphoreType.DMA,
            pltpu.SemaphoreType.DMA,
            pltpu.SemaphoreType.DMA,
            pltpu.SemaphoreType.DMA,
            pltpu.SemaphoreType.DMA,
        ],
    )


def _tc_prep_body(degs_ref, degd_ref, x_ref, m1_ref, nsrc_ref, ndst_ref):
    d_out = degs_ref[0, :, 0:1] + degs_ref[1, :, 0:1] + 1.0
    d_in = degd_ref[0, :, 0:1] + degd_ref[1, :, 0:1] + 1.0
    nsrc = jnp.broadcast_to(lax.rsqrt(jnp.maximum(d_out, 1.0)), (_NPAD, _D))
    ndst = jnp.broadcast_to(lax.rsqrt(jnp.maximum(d_in, 1.0)), (_NPAD, _D))
    nsrc_ref[...] = nsrc
    ndst_ref[...] = ndst
    m1_ref[...] = x_ref[...] * nsrc


_tc_prep = pl.pallas_call(
    _tc_prep_body,
    out_shape=(
        jax.ShapeDtypeStruct((_NPAD, _D), jnp.float32),
        jax.ShapeDtypeStruct((_NPAD, _D), jnp.float32),
        jax.ShapeDtypeStruct((_NPAD, _D), jnp.float32),
    ),
)


def _tc_layer_body(p_ref, m_ref, ndst_ref, nsrc_ref, w_ref, b_ref, out_ref):
    t = (p_ref[0] + p_ref[1] + m_ref[...]) * ndst_ref[...]
    h = jnp.dot(t, w_ref[...], preferred_element_type=jnp.float32,
                precision=lax.Precision.HIGHEST)
    h = jnp.maximum(h + b_ref[...], 0.0)
    rows = lax.broadcasted_iota(jnp.int32, (_NPAD, _D), 0)
    out_ref[...] = jnp.where(rows < _N, h * nsrc_ref[...], 0.0)


_tc_layer = pl.pallas_call(
    _tc_layer_body,
    out_shape=jax.ShapeDtypeStruct((_NPAD, _D), jnp.float32),
)


def _tc_final_body(p_ref, m_ref, ndst_ref, w_ref, b_ref, out_ref):
    t = (p_ref[0] + p_ref[1] + m_ref[...]) * ndst_ref[...]
    h = jnp.dot(t, w_ref[...], preferred_element_type=jnp.float32,
                precision=lax.Precision.HIGHEST)
    h = jnp.maximum(h + b_ref[...], 0.0)
    rows = lax.broadcasted_iota(jnp.int32, (_NPAD, _D), 0)
    h = jnp.where(rows < _N, h, 0.0)
    out_ref[...] = jnp.sum(h, axis=0, keepdims=True) * (1.0 / _N)


_tc_final = pl.pallas_call(
    _tc_final_body,
    out_shape=jax.ShapeDtypeStruct((1, _D), jnp.float32),
)


def kernel(x, edge_index, W1, b1, W2, b2, W3, b3):
    src = edge_index[0]
    dst = edge_index[1]
    pad_idx = _N + (jnp.arange(_EPAD - _E, dtype=jnp.int32) % (_NPAD - _N))
    src3 = jnp.concatenate([src, pad_idx]).reshape(_NW, _CPT, _CHUNK)
    dst3 = jnp.concatenate([dst, pad_idx]).reshape(_NW, _CPT, _CHUNK)

    zeros128 = jnp.zeros((_NPAD, _D), jnp.float32)
    zeros16 = jnp.zeros((_NPAD, 16), jnp.float32)
    ones16 = jnp.ones((_CHUNK, 16), jnp.float32)
    x_pad = jnp.pad(x, ((0, _NPAD - _N), (0, 0)))

    degs, degd = _degree_kernel()(src3, dst3, zeros16, ones16)
    m1, nsrc, ndst = _tc_prep(degs, degd, x_pad)

    b1r = b1.reshape(1, _D)
    b2r = b2.reshape(1, _D)
    b3r = b3.reshape(1, _D)

    scat = _scatter_kernel()
    p1 = scat(m1, src3, dst3, zeros128)
    m2 = _tc_layer(p1, m1, ndst, nsrc, W1, b1r)
    p2 = scat(m2, src3, dst3, zeros128)
    m3 = _tc_layer(p2, m2, ndst, nsrc, W2, b2r)
    p3 = scat(m3, src3, dst3, zeros128)
    out = _tc_final(p3, m3, ndst, W3, b3r)
    return out.reshape(_D)


# speedup vs baseline: 1.0216x; 1.0216x over previous
import jax
import jax.numpy as jnp
from jax import lax
from jax.experimental import pallas as pl
from jax.experimental.pallas import tpu as pltpu
from jax.experimental.pallas import tpu_sc as plsc

_N = 10000
_D = 128
_NPAD = 10240
_E = 320000
_NC = 2
_NS = 16
_NW = _NC * _NS
_CHUNK = 128
_CPT = 80
_CPQ = 16
_QN = 5
_EPAD = _NW * _CPT * _CHUNK
_RPT = _NPAD // _NS

import functools


@functools.cache
def _mesh():
    return plsc.VectorSubcoreMesh(
        core_axis_name="c", subcore_axis_name="s", num_cores=_NC, num_subcores=_NS
    )


def _sc_degree_body(src_hbm, dst_hbm, zero_hbm, ones_hbm, outs_hbm, outd_hbm,
                    srcv, dstv, onesv, accs, accd, ssem0, ssem1):
    cid = lax.axis_index("c")
    sid = lax.axis_index("s")
    wid = sid * _NC + cid
    r0 = sid * _RPT
    pltpu.sync_copy(zero_hbm.at[pl.ds(r0, _RPT)], accs.at[pl.ds(r0, _RPT)])
    pltpu.sync_copy(zero_hbm.at[pl.ds(r0, _RPT)], accd.at[pl.ds(r0, _RPT)])
    pltpu.sync_copy(ones_hbm, onesv)
    pltpu.sync_copy(src_hbm.at[wid], srcv)
    pltpu.sync_copy(dst_hbm.at[wid], dstv)
    plsc.subcore_barrier()

    def body(j, carry):
        pltpu.async_copy(onesv, accs.at[srcv.at[j]], ssem0, add=True)
        pltpu.async_copy(onesv, accd.at[dstv.at[j]], ssem1, add=True)
        return carry

    lax.fori_loop(0, _CPT, body, 0)

    def drain(j, carry):
        pltpu.make_async_copy(onesv, accs.at[srcv.at[j]], ssem0).wait()
        pltpu.make_async_copy(onesv, accd.at[dstv.at[j]], ssem1).wait()
        return carry

    lax.fori_loop(0, _CPT, drain, 0)
    plsc.subcore_barrier()
    pltpu.sync_copy(accs.at[pl.ds(r0, _RPT)], outs_hbm.at[cid, pl.ds(r0, _RPT)])
    pltpu.sync_copy(accd.at[pl.ds(r0, _RPT)], outd_hbm.at[cid, pl.ds(r0, _RPT)])


@functools.cache
def _degree_kernel():
    return pl.kernel(
        _sc_degree_body,
        out_type=(
            jax.ShapeDtypeStruct((_NC, _NPAD, 16), jnp.float32),
            jax.ShapeDtypeStruct((_NC, _NPAD, 16), jnp.float32),
        ),
        mesh=_mesh(),
        scratch_types=[
            pltpu.VMEM((_CPT, _CHUNK), jnp.int32),
            pltpu.VMEM((_CPT, _CHUNK), jnp.int32),
            pltpu.VMEM((_CHUNK, 16), jnp.float32),
            pltpu.VMEM_SHARED((_NPAD, 16), jnp.float32),
            pltpu.VMEM_SHARED((_NPAD, 16), jnp.float32),
            pltpu.SemaphoreType.DMA,
            pltpu.SemaphoreType.DMA,
        ],
        compiler_params=pltpu.CompilerParams(use_tc_tiling_on_sc=False),
    )


def _sc_scatter_body(m_hbm, src_hbm, dst_hbm, zero_hbm, out_hbm,
                     srcv, dstv, rows0, rows1, acc,
                     gsem0, gsem1, ssem0, ssem1, isem0, isem1):
    cid = lax.axis_index("c")
    sid = lax.axis_index("s")
    wid = sid * _NC + cid
    r0 = sid * _RPT
    pltpu.sync_copy(zero_hbm.at[pl.ds(r0, _RPT)], acc.at[pl.ds(r0, _RPT)])
    pltpu.sync_copy(src_hbm.at[wid, pl.ds(0, _CPQ)], srcv.at[0])
    pltpu.sync_copy(dst_hbm.at[wid, pl.ds(0, _CPQ)], dstv.at[0])
    pltpu.async_copy(src_hbm.at[wid, pl.ds(_CPQ, _CPQ)], srcv.at[1], isem0)
    pltpu.async_copy(dst_hbm.at[wid, pl.ds(_CPQ, _CPQ)], dstv.at[1], isem1)
    pltpu.async_copy(m_hbm.at[srcv.at[0, 0]], rows0, gsem0)
    plsc.subcore_barrier()

    for q in range(_QN):
        p = q % 2

        def body(k, carry, p=p):
            j0 = 2 * k
            j1 = j0 + 1
            pltpu.make_async_copy(m_hbm.at[srcv.at[p, j0]], rows0, gsem0).wait()

            @pl.when(k > 0)
            def _():
                pltpu.make_async_copy(rows1, acc.at[dstv.at[p, j1 - 2]], ssem1).wait()

            pltpu.async_copy(m_hbm.at[srcv.at[p, j1]], rows1, gsem1)
            pltpu.async_copy(rows0, acc.at[dstv.at[p, j0]], ssem0, add=True)
            pltpu.make_async_copy(m_hbm.at[srcv.at[p, j1]], rows1, gsem1).wait()
            pltpu.async_copy(rows1, acc.at[dstv.at[p, j1]], ssem1, add=True)
            pltpu.make_async_copy(rows0, acc.at[dstv.at[p, j0]], ssem0).wait()

            @pl.when(k < _CPQ // 2 - 1)
            def _():
                pltpu.async_copy(m_hbm.at[srcv.at[p, j0 + 2]], rows0, gsem0)

            return carry

        lax.fori_loop(0, _CPQ // 2, body, 0)
        if q + 1 < _QN:
            pltpu.make_async_copy(
                src_hbm.at[wid, pl.ds((q + 1) * _CPQ, _CPQ)], srcv.at[1 - p], isem0).wait()
            pltpu.make_async_copy(
                dst_hbm.at[wid, pl.ds((q + 1) * _CPQ, _CPQ)], dstv.at[1 - p], isem1).wait()
            if q + 2 < _QN:
                pltpu.async_copy(
                    src_hbm.at[wid, pl.ds((q + 2) * _CPQ, _CPQ)], srcv.at[p], isem0)
                pltpu.async_copy(
                    dst_hbm.at[wid, pl.ds((q + 2) * _CPQ, _CPQ)], dstv.at[p], isem1)
            pltpu.async_copy(m_hbm.at[srcv.at[1 - p, 0]], rows0, gsem0)
        pltpu.make_async_copy(rows1, acc.at[dstv.at[p, _CPQ - 1]], ssem1).wait()

    plsc.subcore_barrier()
    pltpu.sync_copy(acc.at[pl.ds(r0, _RPT)], out_hbm.at[cid, pl.ds(r0, _RPT)])


@functools.cache
def _scatter_kernel():
    return pl.kernel(
        _sc_scatter_body,
        out_type=jax.ShapeDtypeStruct((_NC, _NPAD, _D), jnp.float32),
        mesh=_mesh(),
        scratch_types=[
            pltpu.VMEM((2, _CPQ, _CHUNK), jnp.int32),
            pltpu.VMEM((2, _CPQ, _CHUNK), jnp.int32),
            pltpu.VMEM((_CHUNK, _D), jnp.float32),
            pltpu.VMEM((_CHUNK, _D), jnp.float32),
            pltpu.VMEM_SHARED((_NPAD, _D), jnp.float32),
            pltpu.SemaphoreType.DMA,
            pltpu.SemaphoreType.DMA,
            pltpu.SemaphoreType.DMA,
            pltpu.SemaphoreType.DMA,
            pltpu.SemaphoreType.DMA,
            pltpu.SemaphoreType.DMA,
        ],
    )


def _tc_prep_body(degs_ref, degd_ref, x_ref, m1_ref, nsrc_ref, ndst_ref):
    d_out = degs_ref[0, :, 0:1] + degs_ref[1, :, 0:1] + 1.0
    d_in = degd_ref[0, :, 0:1] + degd_ref[1, :, 0:1] + 1.0
    nsrc = jnp.broadcast_to(lax.rsqrt(jnp.maximum(d_out, 1.0)), (_NPAD, _D))
    ndst = jnp.broadcast_to(lax.rsqrt(jnp.maximum(d_in, 1.0)), (_NPAD, _D))
    nsrc_ref[...] = nsrc
    ndst_ref[...] = ndst
    m1_ref[...] = x_ref[...] * nsrc


_tc_prep = pl.pallas_call(
    _tc_prep_body,
    out_shape=(
        jax.ShapeDtypeStruct((_NPAD, _D), jnp.float32),
        jax.ShapeDtypeStruct((_NPAD, _D), jnp.float32),
        jax.ShapeDtypeStruct((_NPAD, _D), jnp.float32),
    ),
)


def _tc_layer_body(p_ref, m_ref, ndst_ref, nsrc_ref, w_ref, b_ref, out_ref):
    t = (p_ref[0] + p_ref[1] + m_ref[...]) * ndst_ref[...]
    h = jnp.dot(t, w_ref[...], preferred_element_type=jnp.float32,
                precision=lax.Precision.HIGHEST)
    h = jnp.maximum(h + b_ref[...], 0.0)
    rows = lax.broadcasted_iota(jnp.int32, (_NPAD, _D), 0)
    out_ref[...] = jnp.where(rows < _N, h * nsrc_ref[...], 0.0)


_tc_layer = pl.pallas_call(
    _tc_layer_body,
    out_shape=jax.ShapeDtypeStruct((_NPAD, _D), jnp.float32),
)


def _tc_final_body(p_ref, m_ref, ndst_ref, w_ref, b_ref, out_ref):
    t = (p_ref[0] + p_ref[1] + m_ref[...]) * ndst_ref[...]
    h = jnp.dot(t, w_ref[...], preferred_element_type=jnp.float32,
                precision=lax.Precision.HIGHEST)
    h = jnp.maximum(h + b_ref[...], 0.0)
    rows = lax.broadcasted_iota(jnp.int32, (_NPAD, _D), 0)
    h = jnp.where(rows < _N, h, 0.0)
    out_ref[...] = jnp.sum(h, axis=0, keepdims=True) * (1.0 / _N)


_tc_final = pl.pallas_call(
    _tc_final_body,
    out_shape=jax.ShapeDtypeStruct((1, _D), jnp.float32),
)


def kernel(x, edge_index, W1, b1, W2, b2, W3, b3):
    src = edge_index[0]
    dst = edge_index[1]
    pad_idx = _N + (jnp.arange(_EPAD - _E, dtype=jnp.int32) % (_NPAD - _N))
    src3 = jnp.concatenate([src, pad_idx]).reshape(_NW, _CPT, _CHUNK)
    dst3 = jnp.concatenate([dst, pad_idx]).reshape(_NW, _CPT, _CHUNK)

    zeros128 = jnp.zeros((_NPAD, _D), jnp.float32)
    zeros16 = jnp.zeros((_NPAD, 16), jnp.float32)
    ones16 = jnp.ones((_CHUNK, 16), jnp.float32)
    x_pad = jnp.pad(x, ((0, _NPAD - _N), (0, 0)))

    degs, degd = _degree_kernel()(src3, dst3, zeros16, ones16)
    m1, nsrc, ndst = _tc_prep(degs, degd, x_pad)

    b1r = b1.reshape(1, _D)
    b2r = b2.reshape(1, _D)
    b3r = b3.reshape(1, _D)

    scat = _scatter_kernel()
    p1 = scat(m1, src3, dst3, zeros128)
    m2 = _tc_layer(p1, m1, ndst, nsrc, W1, b1r)
    p2 = scat(m2, src3, dst3, zeros128)
    m3 = _tc_layer(p2, m2, ndst, nsrc, W2, b2r)
    p3 = scat(m3, src3, dst3, zeros128)
    out = _tc_final(p3, m3, ndst, W3, b3r)
    return out.reshape(_D)


# speedup vs baseline: 1.0329x; 1.0110x over previous
import jax
import jax.numpy as jnp
from jax import lax
from jax.experimental import pallas as pl
from jax.experimental.pallas import tpu as pltpu
from jax.experimental.pallas import tpu_sc as plsc

_N = 10000
_D = 128
_NPAD = 10240
_E = 320000
_NC = 2
_NS = 16
_NW = _NC * _NS
_CHUNK = 128
_CPT = 80
_CPQ = 16
_QN = 5
_EPAD = _NW * _CPT * _CHUNK
_RPT = _NPAD // _NS

import functools


@functools.cache
def _mesh():
    return plsc.VectorSubcoreMesh(
        core_axis_name="c", subcore_axis_name="s", num_cores=_NC, num_subcores=_NS
    )


def _sc_degree_body(src_hbm, dst_hbm, zero_hbm, ones_hbm, outs_hbm, outd_hbm,
                    srcv, dstv, onesv, accs, accd, ssem0, ssem1):
    cid = lax.axis_index("c")
    sid = lax.axis_index("s")
    wid = sid * _NC + cid
    r0 = sid * _RPT
    pltpu.sync_copy(zero_hbm.at[pl.ds(r0, _RPT)], accs.at[pl.ds(r0, _RPT)])
    pltpu.sync_copy(zero_hbm.at[pl.ds(r0, _RPT)], accd.at[pl.ds(r0, _RPT)])
    pltpu.sync_copy(ones_hbm, onesv)
    pltpu.sync_copy(src_hbm.at[wid], srcv)
    pltpu.sync_copy(dst_hbm.at[wid], dstv)
    plsc.subcore_barrier()

    def body(j, carry):
        pltpu.async_copy(onesv, accs.at[srcv.at[j]], ssem0, add=True)
        pltpu.async_copy(onesv, accd.at[dstv.at[j]], ssem1, add=True)
        return carry

    lax.fori_loop(0, _CPT, body, 0)

    def drain(j, carry):
        pltpu.make_async_copy(onesv, accs.at[srcv.at[j]], ssem0).wait()
        pltpu.make_async_copy(onesv, accd.at[dstv.at[j]], ssem1).wait()
        return carry

    lax.fori_loop(0, _CPT, drain, 0)
    plsc.subcore_barrier()
    pltpu.sync_copy(accs.at[pl.ds(r0, _RPT)], outs_hbm.at[cid, pl.ds(r0, _RPT)])
    pltpu.sync_copy(accd.at[pl.ds(r0, _RPT)], outd_hbm.at[cid, pl.ds(r0, _RPT)])


@functools.cache
def _degree_kernel():
    return pl.kernel(
        _sc_degree_body,
        out_type=(
            jax.ShapeDtypeStruct((_NC, _NPAD, 16), jnp.float32),
            jax.ShapeDtypeStruct((_NC, _NPAD, 16), jnp.float32),
        ),
        mesh=_mesh(),
        scratch_types=[
            pltpu.VMEM((_CPT, _CHUNK), jnp.int32),
            pltpu.VMEM((_CPT, _CHUNK), jnp.int32),
            pltpu.VMEM((_CHUNK, 16), jnp.float32),
            pltpu.VMEM_SHARED((_NPAD, 16), jnp.float32),
            pltpu.VMEM_SHARED((_NPAD, 16), jnp.float32),
            pltpu.SemaphoreType.DMA,
            pltpu.SemaphoreType.DMA,
        ],
        compiler_params=pltpu.CompilerParams(use_tc_tiling_on_sc=False),
    )


def _sc_scatter_body(m_hbm, src_hbm, dst_hbm, zero_hbm, out_hbm,
                     srcv, dstv, rows0, rows1, acc,
                     gsem0, gsem1, ssem0, ssem1, isem0, isem1, zsem):
    cid = lax.axis_index("c")
    sid = lax.axis_index("s")
    wid = sid * _NC + cid
    r0 = sid * _RPT
    zinit = pltpu.async_copy(
        zero_hbm.at[pl.ds(r0, _RPT)], acc.at[pl.ds(r0, _RPT)], zsem)
    pltpu.sync_copy(src_hbm.at[wid, pl.ds(0, _CPQ)], srcv.at[0])
    pltpu.sync_copy(dst_hbm.at[wid, pl.ds(0, _CPQ)], dstv.at[0])
    pltpu.async_copy(src_hbm.at[wid, pl.ds(_CPQ, _CPQ)], srcv.at[1], isem0)
    pltpu.async_copy(dst_hbm.at[wid, pl.ds(_CPQ, _CPQ)], dstv.at[1], isem1)
    pltpu.async_copy(m_hbm.at[srcv.at[0, 0]], rows0, gsem0)
    zinit.wait()
    plsc.subcore_barrier()

    for q in range(_QN):
        p = q % 2

        def body(k, carry, p=p):
            j0 = 2 * k
            j1 = j0 + 1
            pltpu.make_async_copy(m_hbm.at[srcv.at[p, j0]], rows0, gsem0).wait()

            @pl.when(k > 0)
            def _():
                pltpu.make_async_copy(rows1, acc.at[dstv.at[p, j1 - 2]], ssem1).wait()

            pltpu.async_copy(m_hbm.at[srcv.at[p, j1]], rows1, gsem1)
            pltpu.async_copy(rows0, acc.at[dstv.at[p, j0]], ssem0, add=True)
            pltpu.make_async_copy(m_hbm.at[srcv.at[p, j1]], rows1, gsem1).wait()
            pltpu.async_copy(rows1, acc.at[dstv.at[p, j1]], ssem1, add=True)
            pltpu.make_async_copy(rows0, acc.at[dstv.at[p, j0]], ssem0).wait()

            @pl.when(k < _CPQ // 2 - 1)
            def _():
                pltpu.async_copy(m_hbm.at[srcv.at[p, j0 + 2]], rows0, gsem0)

            return carry

        lax.fori_loop(0, _CPQ // 2, body, 0)
        if q + 1 < _QN:
            pltpu.make_async_copy(
                src_hbm.at[wid, pl.ds((q + 1) * _CPQ, _CPQ)], srcv.at[1 - p], isem0).wait()
            pltpu.make_async_copy(
                dst_hbm.at[wid, pl.ds((q + 1) * _CPQ, _CPQ)], dstv.at[1 - p], isem1).wait()
            if q + 2 < _QN:
                pltpu.async_copy(
                    src_hbm.at[wid, pl.ds((q + 2) * _CPQ, _CPQ)], srcv.at[p], isem0)
                pltpu.async_copy(
                    dst_hbm.at[wid, pl.ds((q + 2) * _CPQ, _CPQ)], dstv.at[p], isem1)
            pltpu.async_copy(m_hbm.at[srcv.at[1 - p, 0]], rows0, gsem0)
        pltpu.make_async_copy(rows1, acc.at[dstv.at[p, _CPQ - 1]], ssem1).wait()

    plsc.subcore_barrier()
    pltpu.sync_copy(acc.at[pl.ds(r0, _RPT)], out_hbm.at[cid, pl.ds(r0, _RPT)])


@functools.cache
def _scatter_kernel():
    return pl.kernel(
        _sc_scatter_body,
        out_type=jax.ShapeDtypeStruct((_NC, _NPAD, _D), jnp.float32),
        mesh=_mesh(),
        scratch_types=[
            pltpu.VMEM((2, _CPQ, _CHUNK), jnp.int32),
            pltpu.VMEM((2, _CPQ, _CHUNK), jnp.int32),
            pltpu.VMEM((_CHUNK, _D), jnp.float32),
            pltpu.VMEM((_CHUNK, _D), jnp.float32),
            pltpu.VMEM_SHARED((_NPAD, _D), jnp.float32),
            pltpu.SemaphoreType.DMA,
            pltpu.SemaphoreType.DMA,
            pltpu.SemaphoreType.DMA,
            pltpu.SemaphoreType.DMA,
            pltpu.SemaphoreType.DMA,
            pltpu.SemaphoreType.DMA,
            pltpu.SemaphoreType.DMA,
        ],
    )


def _tc_prep_body(degs_ref, degd_ref, x_ref, m1_ref, nsrc_ref, ndst_ref):
    d_out = degs_ref[0, :, 0:1] + degs_ref[1, :, 0:1] + 1.0
    d_in = degd_ref[0, :, 0:1] + degd_ref[1, :, 0:1] + 1.0
    nsrc = jnp.broadcast_to(lax.rsqrt(jnp.maximum(d_out, 1.0)), (_NPAD, _D))
    ndst = jnp.broadcast_to(lax.rsqrt(jnp.maximum(d_in, 1.0)), (_NPAD, _D))
    nsrc_ref[...] = nsrc
    ndst_ref[...] = ndst
    m1_ref[...] = x_ref[...] * nsrc


_tc_prep = pl.pallas_call(
    _tc_prep_body,
    out_shape=(
        jax.ShapeDtypeStruct((_NPAD, _D), jnp.float32),
        jax.ShapeDtypeStruct((_NPAD, _D), jnp.float32),
        jax.ShapeDtypeStruct((_NPAD, _D), jnp.float32),
    ),
)


def _tc_layer_body(p_ref, m_ref, ndst_ref, nsrc_ref, w_ref, b_ref, out_ref):
    t = (p_ref[0] + p_ref[1] + m_ref[...]) * ndst_ref[...]
    h = jnp.dot(t, w_ref[...], preferred_element_type=jnp.float32,
                precision=lax.Precision.HIGHEST)
    h = jnp.maximum(h + b_ref[...], 0.0)
    rows = lax.broadcasted_iota(jnp.int32, (_NPAD, _D), 0)
    out_ref[...] = jnp.where(rows < _N, h * nsrc_ref[...], 0.0)


_tc_layer = pl.pallas_call(
    _tc_layer_body,
    out_shape=jax.ShapeDtypeStruct((_NPAD, _D), jnp.float32),
)


def _tc_final_body(p_ref, m_ref, ndst_ref, w_ref, b_ref, out_ref):
    t = (p_ref[0] + p_ref[1] + m_ref[...]) * ndst_ref[...]
    h = jnp.dot(t, w_ref[...], preferred_element_type=jnp.float32,
                precision=lax.Precision.HIGHEST)
    h = jnp.maximum(h + b_ref[...], 0.0)
    rows = lax.broadcasted_iota(jnp.int32, (_NPAD, _D), 0)
    h = jnp.where(rows < _N, h, 0.0)
    out_ref[...] = jnp.sum(h, axis=0, keepdims=True) * (1.0 / _N)


_tc_final = pl.pallas_call(
    _tc_final_body,
    out_shape=jax.ShapeDtypeStruct((1, _D), jnp.float32),
)


def kernel(x, edge_index, W1, b1, W2, b2, W3, b3):
    src = edge_index[0]
    dst = edge_index[1]
    pad_idx = _N + (jnp.arange(_EPAD - _E, dtype=jnp.int32) % (_NPAD - _N))
    src3 = jnp.concatenate([src, pad_idx]).reshape(_NW, _CPT, _CHUNK)
    dst3 = jnp.concatenate([dst, pad_idx]).reshape(_NW, _CPT, _CHUNK)

    zeros128 = jnp.zeros((_NPAD, _D), jnp.float32)
    zeros16 = jnp.zeros((_NPAD, 16), jnp.float32)
    ones16 = jnp.ones((_CHUNK, 16), jnp.float32)
    x_pad = jnp.pad(x, ((0, _NPAD - _N), (0, 0)))

    degs, degd = _degree_kernel()(src3, dst3, zeros16, ones16)
    m1, nsrc, ndst = _tc_prep(degs, degd, x_pad)

    b1r = b1.reshape(1, _D)
    b2r = b2.reshape(1, _D)
    b3r = b3.reshape(1, _D)

    scat = _scatter_kernel()
    p1 = scat(m1, src3, dst3, zeros128)
    m2 = _tc_layer(p1, m1, ndst, nsrc, W1, b1r)
    p2 = scat(m2, src3, dst3, zeros128)
    m3 = _tc_layer(p2, m2, ndst, nsrc, W2, b2r)
    p3 = scat(m3, src3, dst3, zeros128)
    out = _tc_final(p3, m3, ndst, W3, b3r)
    return out.reshape(_D)


# speedup vs baseline: 1.0350x; 1.0020x over previous
import jax
import jax.numpy as jnp
from jax import lax
from jax.experimental import pallas as pl
from jax.experimental.pallas import tpu as pltpu
from jax.experimental.pallas import tpu_sc as plsc

_N = 10000
_D = 128
_NPAD = 10240
_E = 320000
_NC = 2
_NS = 16
_NW = _NC * _NS
_CHUNK = 128
_CPT = 80
_CPQ = 16
_QN = 5
_EPAD = _NW * _CPT * _CHUNK
_RPT = _NPAD // _NS

import functools


@functools.cache
def _mesh():
    return plsc.VectorSubcoreMesh(
        core_axis_name="c", subcore_axis_name="s", num_cores=_NC, num_subcores=_NS
    )


def _sc_degree_body(src_hbm, dst_hbm, zero_hbm, ones_hbm, outs_hbm, outd_hbm,
                    srcv, dstv, onesv, accs, accd, ssem0, ssem1):
    cid = lax.axis_index("c")
    sid = lax.axis_index("s")
    wid = sid * _NC + cid
    r0 = sid * _RPT
    z0 = pltpu.async_copy(
        zero_hbm.at[pl.ds(r0, _RPT)], accs.at[pl.ds(r0, _RPT)], ssem0)
    z1 = pltpu.async_copy(
        zero_hbm.at[pl.ds(r0, _RPT)], accd.at[pl.ds(r0, _RPT)], ssem1)
    pltpu.sync_copy(ones_hbm, onesv)
    pltpu.sync_copy(src_hbm.at[wid], srcv)
    pltpu.sync_copy(dst_hbm.at[wid], dstv)
    z0.wait()
    z1.wait()
    plsc.subcore_barrier()

    def body(j, carry):
        pltpu.async_copy(onesv, accs.at[srcv.at[j]], ssem0, add=True)
        pltpu.async_copy(onesv, accd.at[dstv.at[j]], ssem1, add=True)
        return carry

    lax.fori_loop(0, _CPT, body, 0)

    def drain(j, carry):
        pltpu.make_async_copy(onesv, accs.at[srcv.at[j]], ssem0).wait()
        pltpu.make_async_copy(onesv, accd.at[dstv.at[j]], ssem1).wait()
        return carry

    lax.fori_loop(0, _CPT, drain, 0)
    plsc.subcore_barrier()
    c0 = pltpu.async_copy(
        accs.at[pl.ds(r0, _RPT)], outs_hbm.at[cid, pl.ds(r0, _RPT)], ssem0)
    c1 = pltpu.async_copy(
        accd.at[pl.ds(r0, _RPT)], outd_hbm.at[cid, pl.ds(r0, _RPT)], ssem1)
    c0.wait()
    c1.wait()


@functools.cache
def _degree_kernel():
    return pl.kernel(
        _sc_degree_body,
        out_type=(
            jax.ShapeDtypeStruct((_NC, _NPAD, 16), jnp.float32),
            jax.ShapeDtypeStruct((_NC, _NPAD, 16), jnp.float32),
        ),
        mesh=_mesh(),
        scratch_types=[
            pltpu.VMEM((_CPT, _CHUNK), jnp.int32),
            pltpu.VMEM((_CPT, _CHUNK), jnp.int32),
            pltpu.VMEM((_CHUNK, 16), jnp.float32),
            pltpu.VMEM_SHARED((_NPAD, 16), jnp.float32),
            pltpu.VMEM_SHARED((_NPAD, 16), jnp.float32),
            pltpu.SemaphoreType.DMA,
            pltpu.SemaphoreType.DMA,
        ],
        compiler_params=pltpu.CompilerParams(use_tc_tiling_on_sc=False),
    )


def _sc_scatter_body(m_hbm, src_hbm, dst_hbm, zero_hbm, out_hbm,
                     srcv, dstv, rows0, rows1, acc,
                     gsem0, gsem1, ssem0, ssem1, isem0, isem1, zsem):
    cid = lax.axis_index("c")
    sid = lax.axis_index("s")
    wid = sid * _NC + cid
    r0 = sid * _RPT
    zinit = pltpu.async_copy(
        zero_hbm.at[pl.ds(r0, _RPT)], acc.at[pl.ds(r0, _RPT)], zsem)
    pltpu.sync_copy(src_hbm.at[wid, pl.ds(0, _CPQ)], srcv.at[0])
    pltpu.sync_copy(dst_hbm.at[wid, pl.ds(0, _CPQ)], dstv.at[0])
    pltpu.async_copy(src_hbm.at[wid, pl.ds(_CPQ, _CPQ)], srcv.at[1], isem0)
    pltpu.async_copy(dst_hbm.at[wid, pl.ds(_CPQ, _CPQ)], dstv.at[1], isem1)
    pltpu.async_copy(m_hbm.at[srcv.at[0, 0]], rows0, gsem0)
    zinit.wait()
    plsc.subcore_barrier()

    for q in range(_QN):
        p = q % 2

        def body(k, carry, p=p):
            j0 = 2 * k
            j1 = j0 + 1
            pltpu.make_async_copy(m_hbm.at[srcv.at[p, j0]], rows0, gsem0).wait()

            @pl.when(k > 0)
            def _():
                pltpu.make_async_copy(rows1, acc.at[dstv.at[p, j1 - 2]], ssem1).wait()

            pltpu.async_copy(m_hbm.at[srcv.at[p, j1]], rows1, gsem1)
            pltpu.async_copy(rows0, acc.at[dstv.at[p, j0]], ssem0, add=True)
            pltpu.make_async_copy(m_hbm.at[srcv.at[p, j1]], rows1, gsem1).wait()
            pltpu.async_copy(rows1, acc.at[dstv.at[p, j1]], ssem1, add=True)
            pltpu.make_async_copy(rows0, acc.at[dstv.at[p, j0]], ssem0).wait()

            @pl.when(k < _CPQ // 2 - 1)
            def _():
                pltpu.async_copy(m_hbm.at[srcv.at[p, j0 + 2]], rows0, gsem0)

            return carry

        lax.fori_loop(0, _CPQ // 2, body, 0)
        if q + 1 < _QN:
            pltpu.make_async_copy(
                src_hbm.at[wid, pl.ds((q + 1) * _CPQ, _CPQ)], srcv.at[1 - p], isem0).wait()
            pltpu.make_async_copy(
                dst_hbm.at[wid, pl.ds((q + 1) * _CPQ, _CPQ)], dstv.at[1 - p], isem1).wait()
            if q + 2 < _QN:
                pltpu.async_copy(
                    src_hbm.at[wid, pl.ds((q + 2) * _CPQ, _CPQ)], srcv.at[p], isem0)
                pltpu.async_copy(
                    dst_hbm.at[wid, pl.ds((q + 2) * _CPQ, _CPQ)], dstv.at[p], isem1)
            pltpu.async_copy(m_hbm.at[srcv.at[1 - p, 0]], rows0, gsem0)
        pltpu.make_async_copy(rows1, acc.at[dstv.at[p, _CPQ - 1]], ssem1).wait()

    plsc.subcore_barrier()
    pltpu.sync_copy(acc.at[pl.ds(r0, _RPT)], out_hbm.at[cid, pl.ds(r0, _RPT)])


@functools.cache
def _scatter_kernel():
    return pl.kernel(
        _sc_scatter_body,
        out_type=jax.ShapeDtypeStruct((_NC, _NPAD, _D), jnp.float32),
        mesh=_mesh(),
        scratch_types=[
            pltpu.VMEM((2, _CPQ, _CHUNK), jnp.int32),
            pltpu.VMEM((2, _CPQ, _CHUNK), jnp.int32),
            pltpu.VMEM((_CHUNK, _D), jnp.float32),
            pltpu.VMEM((_CHUNK, _D), jnp.float32),
            pltpu.VMEM_SHARED((_NPAD, _D), jnp.float32),
            pltpu.SemaphoreType.DMA,
            pltpu.SemaphoreType.DMA,
            pltpu.SemaphoreType.DMA,
            pltpu.SemaphoreType.DMA,
            pltpu.SemaphoreType.DMA,
            pltpu.SemaphoreType.DMA,
            pltpu.SemaphoreType.DMA,
        ],
    )


def _tc_prep_body(degs_ref, degd_ref, x_ref, m1_ref, nsrc_ref, ndst_ref):
    d_out = degs_ref[0, :, 0:1] + degs_ref[1, :, 0:1] + 1.0
    d_in = degd_ref[0, :, 0:1] + degd_ref[1, :, 0:1] + 1.0
    nsrc = jnp.broadcast_to(lax.rsqrt(jnp.maximum(d_out, 1.0)), (_NPAD, _D))
    ndst = jnp.broadcast_to(lax.rsqrt(jnp.maximum(d_in, 1.0)), (_NPAD, _D))
    nsrc_ref[...] = nsrc
    ndst_ref[...] = ndst
    m1_ref[...] = x_ref[...] * nsrc


_tc_prep = pl.pallas_call(
    _tc_prep_body,
    out_shape=(
        jax.ShapeDtypeStruct((_NPAD, _D), jnp.float32),
        jax.ShapeDtypeStruct((_NPAD, _D), jnp.float32),
        jax.ShapeDtypeStruct((_NPAD, _D), jnp.float32),
    ),
)


def _tc_layer_body(p_ref, m_ref, ndst_ref, nsrc_ref, w_ref, b_ref, out_ref):
    t = (p_ref[0] + p_ref[1] + m_ref[...]) * ndst_ref[...]
    h = jnp.dot(t, w_ref[...], preferred_element_type=jnp.float32,
                precision=lax.Precision.HIGHEST)
    h = jnp.maximum(h + b_ref[...], 0.0)
    rows = lax.broadcasted_iota(jnp.int32, (_NPAD, _D), 0)
    out_ref[...] = jnp.where(rows < _N, h * nsrc_ref[...], 0.0)


_tc_layer = pl.pallas_call(
    _tc_layer_body,
    out_shape=jax.ShapeDtypeStruct((_NPAD, _D), jnp.float32),
)


def _tc_final_body(p_ref, m_ref, ndst_ref, w_ref, b_ref, out_ref):
    t = (p_ref[0] + p_ref[1] + m_ref[...]) * ndst_ref[...]
    h = jnp.dot(t, w_ref[...], preferred_element_type=jnp.float32,
                precision=lax.Precision.HIGHEST)
    h = jnp.maximum(h + b_ref[...], 0.0)
    rows = lax.broadcasted_iota(jnp.int32, (_NPAD, _D), 0)
    h = jnp.where(rows < _N, h, 0.0)
    out_ref[...] = jnp.sum(h, axis=0, keepdims=True) * (1.0 / _N)


_tc_final = pl.pallas_call(
    _tc_final_body,
    out_shape=jax.ShapeDtypeStruct((1, _D), jnp.float32),
)


def kernel(x, edge_index, W1, b1, W2, b2, W3, b3):
    src = edge_index[0]
    dst = edge_index[1]
    pad_idx = _N + (jnp.arange(_EPAD - _E, dtype=jnp.int32) % (_NPAD - _N))
    src3 = jnp.concatenate([src, pad_idx]).reshape(_NW, _CPT, _CHUNK)
    dst3 = jnp.concatenate([dst, pad_idx]).reshape(_NW, _CPT, _CHUNK)

    zeros128 = jnp.zeros((_NPAD, _D), jnp.float32)
    zeros16 = jnp.zeros((_NPAD, 16), jnp.float32)
    ones16 = jnp.ones((_CHUNK, 16), jnp.float32)
    x_pad = jnp.pad(x, ((0, _NPAD - _N), (0, 0)))

    degs, degd = _degree_kernel()(src3, dst3, zeros16, ones16)
    m1, nsrc, ndst = _tc_prep(degs, degd, x_pad)

    b1r = b1.reshape(1, _D)
    b2r = b2.reshape(1, _D)
    b3r = b3.reshape(1, _D)

    scat = _scatter_kernel()
    p1 = scat(m1, src3, dst3, zeros128)
    m2 = _tc_layer(p1, m1, ndst, nsrc, W1, b1r)
    p2 = scat(m2, src3, dst3, zeros128)
    m3 = _tc_layer(p2, m2, ndst, nsrc, W2, b2r)
    p3 = scat(m3, src3, dst3, zeros128)
    out = _tc_final(p3, m3, ndst, W3, b3r)
    return out.reshape(_D)
